# Initial kernel scaffold; baseline (speedup 1.0000x reference)
import jax
import jax.numpy as jnp
from jax.experimental import pallas as pl


def kernel(x, edge_index, los, table, eps1, w1a, b1a, g1, be1, w1b, b1b, eps2, w2a, b2a, g2, be2, w2b, b2b):
    raise NotImplementedError("write your pallas kernel here")


# speedup vs baseline: 10.1760x; 10.1760x over previous
import functools

import jax
import jax.numpy as jnp
from jax import lax
from jax.experimental import pallas as pl
from jax.experimental.pallas import tpu as pltpu
from jax.experimental.pallas import tpu_sc as plsc

_MESH = plsc.VectorSubcoreMesh(core_axis_name="c", subcore_axis_name="s")
_F32 = jnp.float32


def _embed_call(table, fidx, n):
    per_w = n // 32
    sub = per_w // 2

    @functools.partial(
        pl.kernel,
        out_type=jax.ShapeDtypeStruct((n, 64), _F32),
        mesh=_MESH,
        compiler_params=pltpu.CompilerParams(use_tc_tiling_on_sc=False),
        scratch_types=[
            pltpu.VMEM((sub,), jnp.int32),
            pltpu.VMEM((sub, 64), _F32),
            pltpu.SemaphoreType.DMA,
        ],
    )
    def body(table_hbm, fidx_hbm, out_hbm, idx_v, rows_v, sem):
        cid = lax.axis_index("c")
        sid = lax.axis_index("s")
        base = (sid * 2 + cid) * per_w
        for j in range(2):
            b = base + j * sub
            pltpu.sync_copy(fidx_hbm.at[pl.ds(b, sub)], idx_v)
            pltpu.async_copy(table_hbm.at[idx_v], rows_v, sem).wait()
            pltpu.sync_copy(rows_v, out_hbm.at[pl.ds(b, sub)])

    return body(table, fidx)


def _agg_call(hview, src4, dst, n):
    e = src4.shape[1]
    per_s = e // 16
    csz = 2048
    nsub = per_s // csz
    rslice = n // 16

    @functools.partial(
        pl.kernel,
        out_type=jax.ShapeDtypeStruct((n, 64), _F32),
        mesh=_MESH,
        compiler_params=pltpu.CompilerParams(use_tc_tiling_on_sc=False),
        scratch_types=[
            pltpu.VMEM_SHARED((n, 16), _F32),
            pltpu.VMEM((512, 16), _F32),
            pltpu.VMEM((csz,), jnp.int32),
            pltpu.VMEM((csz,), jnp.int32),
            pltpu.VMEM((csz, 16), _F32),
            pltpu.SemaphoreType.DMA,
        ],
    )
    def body(hview_hbm, src4_hbm, dst_hbm, out_hbm,
             acc, zbuf, srcbuf, dstbuf, rows, sem):
        cid = lax.axis_index("c")
        sid = lax.axis_index("s")

        def zinit(i, carry):
            zbuf[i, :] = jnp.zeros((16,), _F32)
            return carry

        lax.fori_loop(0, 512, zinit, 0)

        for p in range(2):
            c = 2 * p
            for j in range(rslice // 512):
                pltpu.sync_copy(zbuf, acc.at[pl.ds(sid * rslice + j * 512, 512)])
            plsc.subcore_barrier()

            def step(s, carry):
                e0 = sid * per_s + s * csz
                pltpu.sync_copy(src4_hbm.at[c + cid, pl.ds(e0, csz)], srcbuf)
                pltpu.sync_copy(dst_hbm.at[pl.ds(e0, csz)], dstbuf)
                pltpu.async_copy(hview_hbm.at[srcbuf], rows, sem).wait()
                pltpu.sync_copy(rows, acc.at[dstbuf], add=True)
                return carry

            lax.fori_loop(0, nsub, step, 0)
            plsc.subcore_barrier()
            pltpu.sync_copy(
                acc.at[pl.ds(sid * rslice, rslice)],
                out_hbm.at[pl.ds(sid * rslice, rslice),
                           pl.ds(16 * (c + cid), 16)])
            plsc.subcore_barrier()

    return body(hview, src4, dst)


def _dense_body(h_ref, a_ref, ep_ref, wa_ref, ba_ref, g_ref, be_ref,
                wb_ref, bb_ref, o_ref, *, readout):
    z = ep_ref[0, 0] * h_ref[...] + a_ref[...]
    z = jnp.dot(z, wa_ref[...], preferred_element_type=_F32,
                precision=lax.Precision.HIGHEST) + ba_ref[...]
    mu = jnp.mean(z, axis=-1, keepdims=True)
    zc = z - mu
    var = jnp.mean(zc * zc, axis=-1, keepdims=True)
    z = zc * lax.rsqrt(var + 1e-5) * g_ref[...] + be_ref[...]
    z = jnp.maximum(z, 0.0)
    z = jnp.dot(z, wb_ref[...], preferred_element_type=_F32,
                precision=lax.Precision.HIGHEST) + bb_ref[...]
    if readout:
        bn = z.shape[0]
        o_ref[...] = jnp.sum(z.reshape(bn // 32, 32, 64), axis=1)
    else:
        o_ref[...] = z


def _dense_call(h, agg, epsp, wa, ba, g, be, wb, bb, *, readout):
    n = h.shape[0]
    bn = 2048
    grid = (n // bn,)
    row_spec = pl.BlockSpec((bn, 64), lambda i: (i, 0))
    fix = lambda shape: pl.BlockSpec(shape, lambda i: (0, 0))
    if readout:
        out_shape = jax.ShapeDtypeStruct((n // 32, 64), _F32)
        out_spec = pl.BlockSpec((bn // 32, 64), lambda i: (i, 0))
    else:
        out_shape = jax.ShapeDtypeStruct((n, 64), _F32)
        out_spec = row_spec
    return pl.pallas_call(
        functools.partial(_dense_body, readout=readout),
        grid=grid,
        in_specs=[row_spec, row_spec, fix((1, 1)), fix((64, 64)), fix((1, 64)),
                  fix((1, 64)), fix((1, 64)), fix((64, 64)), fix((1, 64))],
        out_specs=out_spec,
        out_shape=out_shape,
    )(h, agg, epsp, wa, ba, g, be, wb, bb)


def kernel(x, edge_index, los, table, eps1, w1a, b1a, g1, be1, w1b, b1b,
           eps2, w2a, b2a, g2, be2, w2b, b2b):
    batch, n_cols = x.shape
    card = table.shape[0] // n_cols
    n = batch * n_cols
    e = edge_index.shape[1]

    offsets = (jnp.arange(n_cols, dtype=x.dtype) * card)[None, :]
    fidx = (x + offsets).reshape(-1)
    src = edge_index[0]
    dst = edge_index[1]
    src4 = src[None, :] * 4 + jnp.arange(4, dtype=jnp.int32)[:, None]

    r = lambda v: v.reshape(1, 64)
    ep1 = (1.0 + eps1).reshape(1, 1).astype(_F32)
    ep2 = (1.0 + eps2).reshape(1, 1).astype(_F32)

    h = _embed_call(table, fidx, n)
    agg = _agg_call(h.reshape(4 * n, 16), src4, dst, n)
    h = _dense_call(h, agg, ep1, w1a, r(b1a), r(g1), r(be1), w1b, r(b1b),
                    readout=False)
    agg = _agg_call(h.reshape(4 * n, 16), src4, dst, n)
    return _dense_call(h, agg, ep2, w2a, r(b2a), r(g2), r(be2), w2b, r(b2b),
                       readout=True)
